# Initial kernel scaffold; baseline (speedup 1.0000x reference)
import jax
import jax.numpy as jnp
from jax.experimental import pallas as pl


def kernel(frame_embeddings, templates, template_classes):
    raise NotImplementedError("write your pallas kernel here")


# speedup vs baseline: 3.1714x; 3.1714x over previous
import jax
import jax.numpy as jnp
from jax.experimental import pallas as pl
from jax.experimental.pallas import tpu as pltpu

_THRESH = (50.0, 100.0, 200.0)
_HWB = 512


def _body(cls_ref, x_ref, t_ref, m0_ref, m1_ref, m2_ref, dmin_ref, pcls_ref,
          xt_ref, xn_ref):
    j = pl.program_id(1)
    n_t = pl.num_programs(1)

    @pl.when(j == 0)
    def _prep():
        xt = jnp.swapaxes(x_ref[...], 1, 2)
        xt_ref[...] = xt
        xn_ref[...] = jnp.sum(xt * xt, axis=1)

    t = jnp.swapaxes(t_ref[...], 1, 2)
    xt = xt_ref[...]
    cross = jnp.sum(xt * t, axis=1)
    tn = jnp.sum(t * t, axis=1)
    dist = (xn_ref[...] + tn) - 2.0 * cross
    cls = cls_ref[j]

    @pl.when(j == 0)
    def _init():
        dmin_ref[...] = dist
        pcls_ref[...] = jnp.full(dist.shape, cls, jnp.int32)

    @pl.when(j != 0)
    def _update():
        prev = dmin_ref[...]
        better = dist < prev
        dmin_ref[...] = jnp.where(better, dist, prev)
        pcls_ref[...] = jnp.where(better, cls, pcls_ref[...])

    @pl.when(j == n_t - 1)
    def _masks():
        d = dmin_ref[...]
        m0_ref[...] = d <= _THRESH[0]
        m1_ref[...] = d <= _THRESH[1]
        m2_ref[...] = d <= _THRESH[2]


def kernel(frame_embeddings, templates, template_classes):
    B, HW, D = frame_embeddings.shape
    T = templates.shape[0]
    n_hw = HW // _HWB

    grid_spec = pltpu.PrefetchScalarGridSpec(
        num_scalar_prefetch=1,
        grid=(n_hw, T),
        in_specs=[
            pl.BlockSpec((B, _HWB, D), lambda i, j, cls: (0, i, 0)),
            pl.BlockSpec((1, _HWB, D), lambda i, j, cls: (j, i, 0)),
        ],
        out_specs=[
            pl.BlockSpec((B, _HWB), lambda i, j, cls: (0, i)) for _ in range(5)
        ],
        scratch_shapes=[
            pltpu.VMEM((B, D, _HWB), jnp.float32),
            pltpu.VMEM((B, _HWB), jnp.float32),
        ],
    )
    out_shapes = (
        jax.ShapeDtypeStruct((B, HW), jnp.bool_),
        jax.ShapeDtypeStruct((B, HW), jnp.bool_),
        jax.ShapeDtypeStruct((B, HW), jnp.bool_),
        jax.ShapeDtypeStruct((B, HW), jnp.float32),
        jax.ShapeDtypeStruct((B, HW), jnp.int32),
    )
    m0, m1, m2, dmin, pcls = pl.pallas_call(
        _body,
        grid_spec=grid_spec,
        out_shape=out_shapes,
        compiler_params=pltpu.CompilerParams(
            dimension_semantics=("parallel", "arbitrary"),
        ),
    )(template_classes, frame_embeddings, templates)
    return m0, m1, m2, dmin, pcls
